# Initial kernel scaffold; baseline (speedup 1.0000x reference)
import jax
import jax.numpy as jnp
from jax.experimental import pallas as pl


def kernel(x, edge_index, W, b, gamma, beta):
    raise NotImplementedError("write your pallas kernel here")


# speedup vs baseline: 38.2960x; 38.2960x over previous
import jax
import jax.numpy as jnp
from jax import lax
from jax.experimental import pallas as pl
from jax.experimental.pallas import tpu as pltpu
from jax.experimental.pallas import tpu_sc as plsc

N = 10000
D = 128
E = 320000
EPS = 1e-5

NC = 2
NS = 16
NW = NC * NS
EPW = E // NW
CH = 125
NCH = EPW // CH
HNCH = NCH // 2
N2 = 10240
ZCH = 120
RPS = N2 // NS
RREM = RPS - (RPS // ZCH) * ZCH
DEG_W = 8
NPAD = 10240
DRPS = NPAD // NS

_sc_mesh = plsc.VectorSubcoreMesh(core_axis_name="c", subcore_axis_name="s")


def _deg_body(dst_hbm, ones_hbm, zeros_hbm, out_hbm, dst_v, val_v, zb_v, deg_sh):
    c = lax.axis_index("c")
    s = lax.axis_index("s")
    w = c * NS + s
    pltpu.sync_copy(zeros_hbm, zb_v)
    pltpu.sync_copy(zb_v, deg_sh.at[pl.ds(s * DRPS, DRPS), :])
    pltpu.sync_copy(ones_hbm, val_v)
    pltpu.sync_copy(dst_hbm.at[w], dst_v)
    plsc.subcore_barrier()

    @pl.loop(0, NCH)
    def _scatter(j):
        pltpu.sync_copy(val_v, deg_sh.at[dst_v.at[j]], add=True)

    plsc.subcore_barrier()
    pltpu.sync_copy(deg_sh.at[pl.ds(s * DRPS, DRPS), :], zb_v)
    pltpu.sync_copy(zb_v, out_hbm.at[c, pl.ds(s * DRPS, DRPS), :])


_deg_kernel = pl.kernel(
    _deg_body,
    out_type=jax.ShapeDtypeStruct((NC, NPAD, DEG_W), jnp.float32),
    mesh=_sc_mesh,
    scratch_types=[
        pltpu.VMEM((NCH, CH), jnp.int32),
        pltpu.VMEM((CH, DEG_W), jnp.float32),
        pltpu.VMEM((DRPS, DEG_W), jnp.float32),
        pltpu.VMEM_SHARED((NPAD, DEG_W), jnp.float32),
    ],
    compiler_params=pltpu.CompilerParams(use_tc_tiling_on_sc=False),
)


def _edge_body(g_hbm, src_hbm, dst_hbm, zrows_hbm, out_hbm,
               src_v, dst_v, rows_a, rows_b, acc_sh, sem_a, sem_b):
    c = lax.axis_index("c")
    s = lax.axis_index("s")
    w = c * NS + s
    pltpu.sync_copy(zrows_hbm, rows_a.at[pl.ds(0, ZCH), :])
    base = s * RPS
    for k in range(RPS // ZCH):
        pltpu.sync_copy(rows_a.at[pl.ds(0, ZCH), :],
                        acc_sh.at[pl.ds(base + k * ZCH, ZCH), :])
    pltpu.sync_copy(rows_a.at[pl.ds(0, RREM), :],
                    acc_sh.at[pl.ds(base + (RPS // ZCH) * ZCH, RREM), :])
    plsc.subcore_barrier()

    for half in range(2):
        pltpu.sync_copy(src_hbm.at[w, pl.ds(half * HNCH, HNCH), :], src_v)
        pltpu.sync_copy(dst_hbm.at[w, pl.ds(half * HNCH, HNCH), :], dst_v)
        pltpu.async_copy(g_hbm.at[src_v.at[0]], rows_a, sem_a)

        @pl.loop(0, HNCH, step=2)
        def _pair(j):
            pltpu.async_copy(g_hbm.at[src_v.at[j + 1]], rows_b, sem_b)
            pltpu.make_async_copy(g_hbm.at[src_v.at[j]], rows_a, sem_a).wait()
            pltpu.sync_copy(rows_a, acc_sh.at[dst_v.at[j]], add=True)

            @pl.when(j + 2 < HNCH)
            def _():
                pltpu.async_copy(g_hbm.at[src_v.at[j + 2]], rows_a, sem_a)

            pltpu.make_async_copy(g_hbm.at[src_v.at[j + 1]], rows_b, sem_b).wait()
            pltpu.sync_copy(rows_b, acc_sh.at[dst_v.at[j + 1]], add=True)

    plsc.subcore_barrier()
    for k in range(RPS // ZCH):
        pltpu.sync_copy(acc_sh.at[pl.ds(base + k * ZCH, ZCH), :],
                        rows_a.at[pl.ds(0, ZCH), :])
        pltpu.sync_copy(rows_a.at[pl.ds(0, ZCH), :],
                        out_hbm.at[c, pl.ds(base + k * ZCH, ZCH), :])
    pltpu.sync_copy(acc_sh.at[pl.ds(base + (RPS // ZCH) * ZCH, RREM), :],
                    rows_a.at[pl.ds(0, RREM), :])
    pltpu.sync_copy(rows_a.at[pl.ds(0, RREM), :],
                    out_hbm.at[c, pl.ds(base + (RPS // ZCH) * ZCH, RREM), :])


_edge_kernel = pl.kernel(
    _edge_body,
    out_type=jax.ShapeDtypeStruct((NC, N2, D), jnp.float32),
    mesh=_sc_mesh,
    scratch_types=[
        pltpu.VMEM((HNCH, CH), jnp.int32),
        pltpu.VMEM((HNCH, CH), jnp.int32),
        pltpu.VMEM((CH, D), jnp.float32),
        pltpu.VMEM((CH, D), jnp.float32),
        pltpu.VMEM_SHARED((N2, D), jnp.float32),
        pltpu.SemaphoreType.DMA,
        pltpu.SemaphoreType.DMA,
    ],
    compiler_params=pltpu.CompilerParams(use_tc_tiling_on_sc=False),
)


_MB = 1000


def _mm_body(x_ref, w_ref, o_ref):
    o_ref[...] = jnp.dot(x_ref[...], w_ref[...],
                         preferred_element_type=jnp.float32)


_mm = pl.pallas_call(
    _mm_body,
    grid=(N // _MB,),
    in_specs=[pl.BlockSpec((_MB, D), lambda i: (i, 0)),
              pl.BlockSpec((D, D), lambda i: (0, 0))],
    out_specs=pl.BlockSpec((_MB, D), lambda i: (i, 0)),
    out_shape=jax.ShapeDtypeStruct((N, D), jnp.float32),
)


def _scale_body(h_ref, degp_ref, g_ref, dinv_ref):
    d = degp_ref[0, :, 0:1] + degp_ref[1, :, 0:1] + 1.0
    dv = lax.rsqrt(d)
    g_ref[...] = h_ref[...] * dv
    dinv_ref[...] = jnp.broadcast_to(dv, (_MB, DEG_W))


_scale = pl.pallas_call(
    _scale_body,
    grid=(N // _MB,),
    in_specs=[pl.BlockSpec((_MB, D), lambda i: (i, 0)),
              pl.BlockSpec((NC, _MB, DEG_W), lambda i: (0, i, 0))],
    out_specs=[pl.BlockSpec((_MB, D), lambda i: (i, 0)),
               pl.BlockSpec((_MB, DEG_W), lambda i: (i, 0))],
    out_shape=[jax.ShapeDtypeStruct((N, D), jnp.float32),
               jax.ShapeDtypeStruct((N, DEG_W), jnp.float32)],
)


def _bn1_body(accp_ref, g_ref, dinv_ref, pre_ref, s1_ref, s2_ref):
    i = pl.program_id(0)
    pre = (accp_ref[0] + accp_ref[1] + g_ref[...]) * dinv_ref[:, 0:1]
    pre_ref[...] = pre

    @pl.when(i == 0)
    def _():
        s1_ref[...] = jnp.zeros_like(s1_ref)
        s2_ref[...] = jnp.zeros_like(s2_ref)

    ps1 = jnp.sum(pre, axis=0, keepdims=True)
    ps2 = jnp.sum(pre * pre, axis=0, keepdims=True)
    s1_ref[...] += jnp.broadcast_to(ps1, (8, D))
    s2_ref[...] += jnp.broadcast_to(ps2, (8, D))


_bn1 = pl.pallas_call(
    _bn1_body,
    grid=(N // _MB,),
    in_specs=[pl.BlockSpec((NC, _MB, D), lambda i: (0, i, 0)),
              pl.BlockSpec((_MB, D), lambda i: (i, 0)),
              pl.BlockSpec((_MB, DEG_W), lambda i: (i, 0))],
    out_specs=[pl.BlockSpec((_MB, D), lambda i: (i, 0)),
               pl.BlockSpec((8, D), lambda i: (0, 0)),
               pl.BlockSpec((8, D), lambda i: (0, 0))],
    out_shape=[jax.ShapeDtypeStruct((N, D), jnp.float32),
               jax.ShapeDtypeStruct((8, D), jnp.float32),
               jax.ShapeDtypeStruct((8, D), jnp.float32)],
)


def _bn2_body(pre_ref, s1_ref, s2_ref, gamma_ref, beta_ref, o_ref):
    mean = s1_ref[0:1, :] * (1.0 / N)
    var = s2_ref[0:1, :] * (1.0 / N) - mean * mean
    scale = lax.rsqrt(var + EPS) * gamma_ref[...]
    o_ref[...] = jnp.maximum((pre_ref[...] - mean) * scale + beta_ref[...], 0.0)


_bn2 = pl.pallas_call(
    _bn2_body,
    grid=(N // _MB,),
    in_specs=[pl.BlockSpec((_MB, D), lambda i: (i, 0)),
              pl.BlockSpec((8, D), lambda i: (0, 0)),
              pl.BlockSpec((8, D), lambda i: (0, 0)),
              pl.BlockSpec((1, D), lambda i: (0, 0)),
              pl.BlockSpec((1, D), lambda i: (0, 0))],
    out_specs=pl.BlockSpec((_MB, D), lambda i: (i, 0)),
    out_shape=jax.ShapeDtypeStruct((N, D), jnp.float32),
)


def kernel(x, edge_index, W, b, gamma, beta):
    del b
    src = edge_index[0].astype(jnp.int32).reshape(NW, NCH, CH)
    dst = edge_index[1].astype(jnp.int32).reshape(NW, NCH, CH)
    ones8 = jnp.ones((CH, DEG_W), jnp.float32)
    zeros8 = jnp.zeros((DRPS, DEG_W), jnp.float32)
    zrows = jnp.zeros((ZCH, D), jnp.float32)

    degp = _deg_kernel(dst, ones8, zeros8)
    h = _mm(x, W)
    g, dinv8 = _scale(h, degp)
    accp = _edge_kernel(g, src, dst, zrows)
    pre, s1, s2 = _bn1(accp, g, dinv8)
    return _bn2(pre, s1, s2, gamma.reshape(1, D), beta.reshape(1, D))
